# Initial kernel scaffold; baseline (speedup 1.0000x reference)
import jax
import jax.numpy as jnp
from jax.experimental import pallas as pl


def kernel(real, imag, edge_index, W1, b1, W2, b2, Cw, Cb):
    raise NotImplementedError("write your pallas kernel here")


# speedup vs baseline: 1.0338x; 1.0338x over previous
import functools

import jax
import jax.numpy as jnp
import numpy as np
from jax.experimental import pallas as pl
from jax.experimental.pallas import tpu as pltpu

_Q = 0.25
_LAMBDA_MAX = 2.0


def _edge_weights(edge_index, n):
    src, dst = edge_index[0], edge_index[1]
    nonself = src != dst
    r = jnp.concatenate([src, dst])
    c = jnp.concatenate([dst, src])
    sgn = jnp.concatenate([jnp.where(nonself, 1, 0), jnp.where(nonself, -1, 0)]).astype(jnp.int32)
    valid = jnp.concatenate([nonself, nonself]).astype(jnp.float32)
    key = r * n + c
    D = jnp.zeros((n * n,), jnp.int32).at[key].add(sgn, mode="drop")
    m4 = jnp.bitwise_and(D[key], 3)
    deg = jnp.zeros((n,), jnp.float32).at[r].add(0.5 * valid)
    dinv = jnp.where(deg > 0, jax.lax.rsqrt(jnp.where(deg > 0, deg, 1.0)), 0.0)
    cosv = jnp.where(m4 == 0, 1.0, jnp.where(m4 == 2, -1.0, 0.0))
    sinv = jnp.where(m4 == 1, 1.0, jnp.where(m4 == 3, -1.0, 0.0))
    scale = 2.0 / _LAMBDA_MAX
    base = (-0.5 * scale) * dinv[r] * dinv[c] * valid
    return r, c, base * cosv, base * sinv


def _lap_mm(xr, xi, r, c, wr, wi):
    gr = xr[c]
    gi = xi[c]
    outr = jnp.zeros_like(xr).at[r].add(wr[:, None] * gr - wi[:, None] * gi)
    outi = jnp.zeros_like(xi).at[r].add(wr[:, None] * gi + wi[:, None] * gr)
    return outr, outi


_BN = 1000


def _conv_body(x0r_ref, x1r_ref, t2r_ref, x0i_ref, x1i_ref, t2i_ref,
               w_ref, b_ref, outr_ref, outi_ref, *, relu):
    w0, w1, w2 = w_ref[0], w_ref[1], w_ref[2]
    x0r = x0r_ref[...]
    x0i = x0i_ref[...]
    dot = functools.partial(jnp.dot, preferred_element_type=jnp.float32)
    outr = dot(x0r, w0) + dot(x1r_ref[...], w1) + dot(2.0 * t2r_ref[...] - x0r, w2) + b_ref[...]
    outi = dot(x0i, w0) + dot(x1i_ref[...], w1) + dot(2.0 * t2i_ref[...] - x0i, w2) + b_ref[...]
    if relu:
        mask = (outr >= 0).astype(jnp.float32)
        outr = mask * outr
        outi = mask * outi
    outr_ref[...] = outr
    outi_ref[...] = outi


def _conv_combine(x0r, x1r, t2r, x0i, x1i, t2i, W, b, relu):
    n, f = x0r.shape
    h = W.shape[2]
    xspec = pl.BlockSpec((_BN, f), lambda i: (i, 0))
    wspec = pl.BlockSpec((3, f, h), lambda i: (0, 0, 0))
    bspec = pl.BlockSpec((h,), lambda i: (0,))
    ospec = pl.BlockSpec((_BN, h), lambda i: (i, 0))
    return pl.pallas_call(
        functools.partial(_conv_body, relu=relu),
        grid=(n // _BN,),
        in_specs=[xspec, xspec, xspec, xspec, xspec, xspec, wspec, bspec],
        out_specs=[ospec, ospec],
        out_shape=[jax.ShapeDtypeStruct((n, h), jnp.float32)] * 2,
    )(x0r, x1r, t2r, x0i, x1i, t2i, W, b)


def _head_body(xr_ref, xi_ref, wr_ref, wi_ref, b_ref, out_ref):
    dot = functools.partial(jnp.dot, preferred_element_type=jnp.float32)
    logits = dot(xr_ref[...], wr_ref[...]) + dot(xi_ref[...], wi_ref[...]) + b_ref[...]
    m = jnp.max(logits, axis=-1, keepdims=True)
    z = logits - m
    lse = jnp.log(jnp.sum(jnp.exp(z), axis=-1, keepdims=True))
    out_ref[...] = z - lse


def _head(xr, xi, Cw, Cb):
    n, h = xr.shape
    cdim = Cw.shape[0]
    wrT = Cw[:, :h].T
    wiT = Cw[:, h:].T
    return pl.pallas_call(
        _head_body,
        grid=(n // _BN,),
        in_specs=[
            pl.BlockSpec((_BN, h), lambda i: (i, 0)),
            pl.BlockSpec((_BN, h), lambda i: (i, 0)),
            pl.BlockSpec((h, cdim), lambda i: (0, 0)),
            pl.BlockSpec((h, cdim), lambda i: (0, 0)),
            pl.BlockSpec((cdim,), lambda i: (0,)),
        ],
        out_specs=pl.BlockSpec((_BN, cdim), lambda i: (i, 0)),
        out_shape=jax.ShapeDtypeStruct((n, cdim), jnp.float32),
    )(xr, xi, wrT, wiT, Cb)


def kernel(real, imag, edge_index, W1, b1, W2, b2, Cw, Cb):
    n = real.shape[0]
    r, c, wr, wi = _edge_weights(edge_index, n)

    def conv(xr, xi, W, b, relu):
        t1r, t1i = _lap_mm(xr, xi, r, c, wr, wi)
        t2r, t2i = _lap_mm(t1r, t1i, r, c, wr, wi)
        return _conv_combine(xr, t1r, t2r, xi, t1i, t2i, W, b, relu)

    xr, xi = conv(real, imag, W1, b1, relu=True)
    xr, xi = conv(xr, xi, W2, b2, relu=True)
    return _head(xr, xi, Cw, Cb)


# speedup vs baseline: 2.1039x; 2.0352x over previous
import functools

import jax
import jax.numpy as jnp
from jax import lax
from jax.experimental import pallas as pl
from jax.experimental.pallas import tpu as pltpu
from jax.experimental.pallas import tpu_sc as plsc

_LAMBDA_MAX = 2.0

_N = 10000
_E = 160000
_M = 2 * _E
_NW = 16
_B = 128
_NB = 160
_MW = _NB * _B
_MPAD = _NW * _MW
_FC = 32
_NP = 8
_NA = 10240
_RT = _NA // 16
_RZ = 32


def _edge_weights(edge_index, n):
    src, dst = edge_index[0], edge_index[1]
    nonself = src != dst
    r = jnp.concatenate([src, dst])
    c = jnp.concatenate([dst, src])
    sgn = jnp.concatenate([jnp.where(nonself, 1, 0), jnp.where(nonself, -1, 0)]).astype(jnp.int32)
    valid = jnp.concatenate([nonself, nonself]).astype(jnp.float32)
    key = r * n + c
    D = jnp.zeros((n * n,), jnp.int32).at[key].add(sgn, mode="drop")
    m4 = jnp.bitwise_and(D[key], 3)
    deg = jnp.zeros((n,), jnp.float32).at[r].add(0.5 * valid)
    dinv = jnp.where(deg > 0, lax.rsqrt(jnp.where(deg > 0, deg, 1.0)), 0.0)
    cosv = jnp.where(m4 == 0, 1.0, jnp.where(m4 == 2, -1.0, 0.0))
    sinv = jnp.where(m4 == 1, 1.0, jnp.where(m4 == 3, -1.0, 0.0))
    scale = 2.0 / _LAMBDA_MAX
    base = (-0.5 * scale) * dinv[r] * dinv[c] * valid
    wr = base * cosv
    wi = base * sinv
    npad = _MPAD - _M
    pad_idx = jnp.arange(npad, dtype=jnp.int32) % n
    pz = jnp.zeros((npad,), jnp.float32)
    cp = jnp.concatenate([c, pad_idx])
    rm = jnp.concatenate([r, pad_idx]).reshape(_NW * _NB, _B)
    idx4 = (cp[None, :] * _NP + jnp.arange(_NP, dtype=jnp.int32)[:, None])
    idx4 = idx4.reshape(_NP * _NW * _NB, _B)
    wri = jnp.stack([jnp.concatenate([wr, pz]).reshape(_NW * _NB, _B),
                     jnp.concatenate([wi, pz]).reshape(_NW * _NB, _B)], axis=1)
    return idx4, rm, wri


def _lap_body(xc_hbm, idx4_hbm, rm_hbm, wri_hbm, out_hbm,
              mi, mr, g0, g1, t0, t1, wb0, wb1, acc, zb,
              sg0, sg1, ss0, ss1):
    sid = lax.axis_index("s")
    wid = sid

    pltpu.sync_copy(rm_hbm.at[pl.ds(wid * _NB, _NB)], mr)

    def _zf(i, _):
        for k in range(2 * _FC // 16):
            zb[i, pl.ds(k * 16, 16)] = jnp.zeros((16,), jnp.float32)
        return 0
    lax.fori_loop(0, _RZ, _zf, 0)

    def _issue_gather(j, g, wb, sem):
        pltpu.async_copy(xc_hbm.at[mi.at[j]], g, sem)
        pltpu.async_copy(wri_hbm.at[wid * _NB + j], wb, sem)

    def _wait_gather(g, wb, sem):
        pltpu.make_async_copy(xc_hbm.at[pl.ds(0, _B)], g, sem).wait()
        pltpu.make_async_copy(wri_hbm.at[0], wb, sem).wait()

    def _issue_scatter(j, t, sem):
        pltpu.async_copy(t, acc.at[mr.at[j]], sem, add=True)

    def _wait_scatter(t, sem):
        pltpu.make_async_copy(xc_hbm.at[pl.ds(0, _B)], t, sem).wait()

    def _compute(g, wb, t):
        def _e16(e16, _):
            wr16 = wb[0, pl.ds(e16 * 16, 16)]
            wi16 = wb[1, pl.ds(e16 * 16, 16)]
            for l in range(16):
                row = e16 * 16 + l
                wrs = wr16[l]
                wis = wi16[l]
                for f in range(_FC // 16):
                    a = g[row, pl.ds(f * 16, 16)]
                    b = g[row, pl.ds(_FC + f * 16, 16)]
                    t[row, pl.ds(f * 16, 16)] = wrs * a - wis * b
                    t[row, pl.ds(_FC + f * 16, 16)] = wrs * b + wis * a
            return 0
        lax.fori_loop(0, _B // 16, _e16, 0)

    for p in range(_NP):
        for z in range(_RT // _RZ):
            pltpu.sync_copy(zb, acc.at[pl.ds(sid * _RT + z * _RZ, _RZ)])
        plsc.subcore_barrier()

        pltpu.sync_copy(idx4_hbm.at[pl.ds((p * _NW + wid) * _NB, _NB)], mi)

        _issue_gather(0, g0, wb0, sg0)

        def _tbody(t_, _):
            j0 = 2 * t_
            j1 = 2 * t_ + 1
            _issue_gather(j1, g1, wb1, sg1)
            _wait_gather(g0, wb0, sg0)

            @pl.when(t_ > 0)
            def _():
                _wait_scatter(t0, ss0)
            _compute(g0, wb0, t0)
            _issue_scatter(j0, t0, ss0)

            @pl.when(t_ + 1 < _NB // 2)
            def _():
                _issue_gather(j0 + 2, g0, wb0, sg0)
            _wait_gather(g1, wb1, sg1)

            @pl.when(t_ > 0)
            def _():
                _wait_scatter(t1, ss1)
            _compute(g1, wb1, t1)
            _issue_scatter(j1, t1, ss1)
            return 0
        lax.fori_loop(0, _NB // 2, _tbody, 0)
        _wait_scatter(t0, ss0)
        _wait_scatter(t1, ss1)
        plsc.subcore_barrier()

        obase = p * _NA + sid * _RT
        for z in range(_RT // _RZ):
            pltpu.sync_copy(acc.at[pl.ds(sid * _RT + z * _RZ, _RZ)],
                            out_hbm.at[pl.ds(obase + z * _RZ, _RZ)])
        plsc.subcore_barrier()


def _lap_sc():
    mesh = plsc.VectorSubcoreMesh(core_axis_name="c", subcore_axis_name="s",
                                  num_cores=1)

    def run(xc_, idx4, rm, wri):
        f = pl.kernel(
            _lap_body,
            mesh=mesh,
            compiler_params=pltpu.CompilerParams(use_tc_tiling_on_sc=False),
            out_type=jax.ShapeDtypeStruct((_NP * _NA, 2 * _FC), jnp.float32),
            scratch_types=[
                pltpu.VMEM((_NB, _B), jnp.int32),
                pltpu.VMEM((_NB, _B), jnp.int32),
                pltpu.VMEM((_B, 2 * _FC), jnp.float32),
                pltpu.VMEM((_B, 2 * _FC), jnp.float32),
                pltpu.VMEM((_B, 2 * _FC), jnp.float32),
                pltpu.VMEM((_B, 2 * _FC), jnp.float32),
                pltpu.VMEM((2, _B), jnp.float32),
                pltpu.VMEM((2, _B), jnp.float32),
                pltpu.VMEM_SHARED((_NA, 2 * _FC), jnp.float32),
                pltpu.VMEM((_RZ, 2 * _FC), jnp.float32),
                pltpu.SemaphoreType.DMA,
                pltpu.SemaphoreType.DMA,
                pltpu.SemaphoreType.DMA,
                pltpu.SemaphoreType.DMA,
            ],
        )
        return f(xc_, idx4, rm, wri)

    return run


_BN = 1000


def _split_blocks(xc):
    w = 2 * _FC
    xr = jnp.concatenate([xc[:, w * q:w * q + _FC] for q in range(_NP)], axis=1)
    xi = jnp.concatenate([xc[:, w * q + _FC:w * q + w] for q in range(_NP)], axis=1)
    return xr, xi


def _merge_blocks(xr, xi):
    return jnp.concatenate(
        [jnp.concatenate([xr[:, _FC * q:_FC * q + _FC], xi[:, _FC * q:_FC * q + _FC]], axis=1)
         for q in range(_NP)], axis=1)


def _combine_body(p_ref, out_ref):
    p = p_ref[...]
    out_ref[...] = jnp.concatenate([p[q] for q in range(_NP)], axis=-1)


def _combine(p):
    return pl.pallas_call(
        _combine_body,
        grid=(_N // _BN,),
        in_specs=[pl.BlockSpec((_NP, _BN, 2 * _FC), lambda i: (0, i, 0))],
        out_specs=pl.BlockSpec((_BN, _NP * 2 * _FC), lambda i: (i, 0)),
        out_shape=jax.ShapeDtypeStruct((_N, _NP * 2 * _FC), jnp.float32),
    )(p)


def _conv_body(x0c_ref, t1c_ref, t2c_ref, w_ref, b_ref, outc_ref):
    x0r, x0i = _split_blocks(x0c_ref[...])
    t1r, t1i = _split_blocks(t1c_ref[...])
    t2r, t2i = _split_blocks(t2c_ref[...])
    w0, w1, w2 = w_ref[0], w_ref[1], w_ref[2]
    dot = functools.partial(jnp.dot, preferred_element_type=jnp.float32)
    outr = dot(x0r, w0) + dot(t1r, w1) + dot(2.0 * t2r - x0r, w2) + b_ref[...]
    outi = dot(x0i, w0) + dot(t1i, w1) + dot(2.0 * t2i - x0i, w2) + b_ref[...]
    mask = (outr >= 0).astype(jnp.float32)
    outr = mask * outr
    outi = mask * outi
    outc_ref[...] = _merge_blocks(outr, outi)


def _conv_combine(x0c, t1c, t2c, W, b):
    h = W.shape[2]
    xspec = pl.BlockSpec((_BN, 2 * h), lambda i: (i, 0))
    return pl.pallas_call(
        _conv_body,
        grid=(_N // _BN,),
        in_specs=[
            xspec, xspec, xspec,
            pl.BlockSpec((3, W.shape[1], h), lambda i: (0, 0, 0)),
            pl.BlockSpec((h,), lambda i: (0,)),
        ],
        out_specs=pl.BlockSpec((_BN, 2 * h), lambda i: (i, 0)),
        out_shape=jax.ShapeDtypeStruct((_N, 2 * h), jnp.float32),
    )(x0c, t1c, t2c, W, b)


def _head_body(xc_ref, wr_ref, wi_ref, b_ref, out_ref):
    xr, xi = _split_blocks(xc_ref[...])
    dot = functools.partial(jnp.dot, preferred_element_type=jnp.float32)
    logits = dot(xr, wr_ref[...]) + dot(xi, wi_ref[...]) + b_ref[...]
    m = jnp.max(logits, axis=-1, keepdims=True)
    z = logits - m
    lse = jnp.log(jnp.sum(jnp.exp(z), axis=-1, keepdims=True))
    out_ref[...] = z - lse


def _head(xc, Cw, Cb):
    h = Cw.shape[1] // 2
    cdim = Cw.shape[0]
    return pl.pallas_call(
        _head_body,
        grid=(_N // _BN,),
        in_specs=[
            pl.BlockSpec((_BN, 2 * h), lambda i: (i, 0)),
            pl.BlockSpec((h, cdim), lambda i: (0, 0)),
            pl.BlockSpec((h, cdim), lambda i: (0, 0)),
            pl.BlockSpec((cdim,), lambda i: (0,)),
        ],
        out_specs=pl.BlockSpec((_BN, cdim), lambda i: (i, 0)),
        out_shape=jax.ShapeDtypeStruct((_N, cdim), jnp.float32),
    )(xc, Cw[:, :h].T, Cw[:, h:].T, Cb)


def kernel(real, imag, edge_index, W1, b1, W2, b2, Cw, Cb):
    n = real.shape[0]
    idx4, rm, wri = _edge_weights(edge_index, n)
    lap_run = _lap_sc()

    def lap(xc):
        p = lap_run(xc.reshape(_NP * n, 2 * _FC), idx4, rm, wri)
        return p.reshape(_NP, _NA, 2 * _FC)

    oz = (edge_index[0, 0] & 0).astype(jnp.int32)
    Ws = jnp.stack([W1, W2])
    bs = jnp.stack([b1, b2])

    def layer(li, xc):
        def lapstep(k, st):
            cur, t1c, t2c = st
            t = _combine(lap(cur))
            t1c = jnp.where(k == 0, t, t1c)
            return t, t1c, t
        zc = jnp.zeros_like(xc)
        _, t1c, t2c = lax.fori_loop(0, 2 + oz, lapstep, (xc, zc, zc))
        W = lax.dynamic_index_in_dim(Ws, li, keepdims=False)
        b = lax.dynamic_index_in_dim(bs, li, keepdims=False)
        return _conv_combine(xc, t1c, t2c, W, b)

    x0c = jnp.concatenate(
        [jnp.concatenate([real[:, _FC * q:_FC * q + _FC], imag[:, _FC * q:_FC * q + _FC]], axis=1)
         for q in range(_NP)], axis=1)
    xc = lax.fori_loop(0, 2 + oz, layer, x0c)
    return _head(xc, Cw, Cb)
